# baseline (speedup 1.0000x reference)
import functools

import jax
import jax.numpy as jnp
from jax import lax
from jax.experimental import pallas as pl
from jax.experimental.pallas import tpu as pltpu
from jax.experimental.pallas import tpu_sc as plsc

_N = 10000
_E = 320000
_D = 128

_L = 16
_NC = 2
_NS = 16
_NW = _NC * _NS
_EPW = _E // _NW
_C = 112
_EPW2 = 10080
_NCHUNK = _EPW2 // _C
_NP = 10240
_RPT = _NP // _NS

_BM = 1000


def _mm2_body(x_ref, wa_ref, wb_ref, oa_ref, ob_ref):
    x = x_ref[...]
    oa_ref[...] = jnp.dot(x, wa_ref[...], preferred_element_type=jnp.float32)
    p = jnp.dot(x, wb_ref[...], preferred_element_type=jnp.float32)
    ob_ref[...] = p.astype(jnp.bfloat16)


def _mm2(x, wa, wb):
    n = x.shape[0]
    return pl.pallas_call(
        _mm2_body,
        grid=(n // _BM,),
        in_specs=[
            pl.BlockSpec((_BM, _D), lambda i: (i, 0)),
            pl.BlockSpec((_D, _D), lambda i: (0, 0)),
            pl.BlockSpec((_D, _D), lambda i: (0, 0)),
        ],
        out_specs=[
            pl.BlockSpec((_BM, _D), lambda i: (i, 0)),
            pl.BlockSpec((_BM, _D), lambda i: (i, 0)),
        ],
        out_shape=[
            jax.ShapeDtypeStruct((n, _D), jnp.float32),
            jax.ShapeDtypeStruct((_NP, _D), jnp.bfloat16),
        ],
    )(x, wa, wb)


def _combine_mm2_body(s_ref, agg_ref, b_ref, wa_ref, wb_ref, oa_ref, ob_ref):
    h = s_ref[...] + agg_ref[0] + agg_ref[1] + b_ref[...]
    h = jnp.maximum(h, 0.0)
    oa_ref[...] = jnp.dot(h, wa_ref[...], preferred_element_type=jnp.float32)
    p = jnp.dot(h, wb_ref[...], preferred_element_type=jnp.float32)
    ob_ref[...] = p.astype(jnp.bfloat16)


def _combine_mm2(s, agg, b, wa, wb):
    n = s.shape[0]
    return pl.pallas_call(
        _combine_mm2_body,
        grid=(n // _BM,),
        in_specs=[
            pl.BlockSpec((_BM, _D), lambda i: (i, 0)),
            pl.BlockSpec((2, _BM, _D), lambda i: (0, i, 0)),
            pl.BlockSpec((1, _D), lambda i: (0, 0)),
            pl.BlockSpec((_D, _D), lambda i: (0, 0)),
            pl.BlockSpec((_D, _D), lambda i: (0, 0)),
        ],
        out_specs=[
            pl.BlockSpec((_BM, _D), lambda i: (i, 0)),
            pl.BlockSpec((_BM, _D), lambda i: (i, 0)),
        ],
        out_shape=[
            jax.ShapeDtypeStruct((n, _D), jnp.float32),
            jax.ShapeDtypeStruct((_NP, _D), jnp.bfloat16),
        ],
    )(s, agg, b.reshape(1, _D), wa, wb)


def _final_body(s_ref, agg_ref, b_ref, o_ref):
    o_ref[...] = s_ref[...] + agg_ref[0] + agg_ref[1] + b_ref[...]


def _final(s, agg, b):
    n = s.shape[0]
    return pl.pallas_call(
        _final_body,
        grid=(n // _BM,),
        in_specs=[
            pl.BlockSpec((_BM, _D), lambda i: (i, 0)),
            pl.BlockSpec((2, _BM, _D), lambda i: (0, i, 0)),
            pl.BlockSpec((1, _D), lambda i: (0, 0)),
        ],
        out_specs=pl.BlockSpec((_BM, _D), lambda i: (i, 0)),
        out_shape=jax.ShapeDtypeStruct((n, _D), jnp.float32),
    )(s, agg, b.reshape(1, _D))


_mesh = plsc.VectorSubcoreMesh(core_axis_name="c", subcore_axis_name="s")


@functools.partial(
    pl.kernel,
    mesh=_mesh,
    compiler_params=pltpu.CompilerParams(
        needs_layout_passes=False, use_tc_tiling_on_sc=False),
    out_type=jax.ShapeDtypeStruct((2, _NP, _D), jnp.float32),
    scratch_types=[
        pltpu.VMEM((3, _C), jnp.int32),
        pltpu.VMEM((3, _C), jnp.int32),
        pltpu.VMEM((3, _C), jnp.float32),
        pltpu.VMEM((_C, _D), jnp.bfloat16),
        pltpu.VMEM((_C, _D), jnp.bfloat16),
        pltpu.VMEM((_C, _D), jnp.float32),
        pltpu.VMEM((_C, _D), jnp.float32),
        pltpu.VMEM_SHARED((_NP, _D), jnp.float32),
        pltpu.SemaphoreType.DMA,
        pltpu.SemaphoreType.DMA,
        pltpu.SemaphoreType.DMA,
        pltpu.SemaphoreType.DMA,
        pltpu.SemaphoreType.DMA,
        pltpu.SemaphoreType.DMA,
        pltpu.SemaphoreType.DMA,
    ],
)
def _sc_scatter(p_hbm, src_hbm, dst_hbm, attr_hbm, zeros_hbm, out_hbm,
                srcr, dstr, attrr, rbf0, rbf1, rows0, rows1, agg_sh,
                esem0, esem1, esem2, gsem0, gsem1, ssem0, ssem1):
    c = lax.axis_index("c")
    s = lax.axis_index("s")
    wid = s * _NC + c
    row0 = s * _RPT

    pltpu.sync_copy(zeros_hbm.at[pl.ds(row0, _RPT)], agg_sh.at[pl.ds(row0, _RPT)])
    plsc.subcore_barrier()

    base = pl.multiple_of(wid * _EPW2, 8)
    esem = (esem0, esem1, esem2)
    rbf = (rbf0, rbf1)
    rows = (rows0, rows1)
    gsem = (gsem0, gsem1)
    ssem = (ssem0, ssem1)

    def issue_edges(ck, e):
        off = pl.multiple_of(base + ck * _C, 8)
        pltpu.async_copy(src_hbm.at[pl.ds(off, _C)], srcr.at[e], esem[e])
        pltpu.async_copy(dst_hbm.at[pl.ds(off, _C)], dstr.at[e], esem[e])
        pltpu.async_copy(attr_hbm.at[pl.ds(off, _C)], attrr.at[e], esem[e])

    def wait_edges(e):
        pltpu.make_async_copy(src_hbm.at[pl.ds(0, _C)], srcr.at[e], esem[e]).wait()
        pltpu.make_async_copy(dst_hbm.at[pl.ds(0, _C)], dstr.at[e], esem[e]).wait()
        pltpu.make_async_copy(attr_hbm.at[pl.ds(0, _C)], attrr.at[e], esem[e]).wait()

    def issue_gather(e, r):
        pltpu.async_copy(p_hbm.at[srcr.at[e]], rbf[r], gsem[r])

    def wait_gather(e, r):
        pltpu.make_async_copy(p_hbm.at[srcr.at[e]], rbf[r], gsem[r]).wait()

    def issue_scatter(e, r):
        pltpu.async_copy(rows[r], agg_sh.at[dstr.at[e]], ssem[r], add=True)

    def wait_scatter(e, r):
        pltpu.make_async_copy(rows[r], agg_sh.at[dstr.at[e]], ssem[r]).wait()

    def scale(e, r):
        av = attrr.at[e]
        bv = rbf[r]
        rv = rows[r]

        def row_body(i, rcarry):
            a = plsc.load_gather(av, [jnp.full((_L,), i, jnp.int32)])
            for t in range(_D // (2 * _L)):
                v = bv[i, pl.ds(2 * _L * t, 2 * _L)]
                lo, hi = plsc.unpack(v, format=plsc.PackFormat.INTERLEAVED)
                rv[i, pl.ds(_L * t, _L)] = lo * a
                rv[i, pl.ds(_D // 2 + _L * t, _L)] = hi * a
            return rcarry

        lax.fori_loop(0, _C, row_body, 0, unroll=2)

    issue_edges(0, 0)
    issue_edges(1, 1)
    issue_edges(2, 2)
    wait_edges(0)
    issue_gather(0, 0)

    def six_body(t, carry):
        k6 = t * 6
        for j in range(6):
            e = j % 3
            r = j % 2
            e_prev = (j - 1) % 3
            e_next = (j + 1) % 3
            r_prev = (j - 1) % 2
            k = k6 + j

            wait_gather(e, r)
            scale(e, r)
            issue_scatter(e, r)

            @pl.when(k >= 1)
            def _(e_prev=e_prev, r_prev=r_prev):
                wait_scatter(e_prev, r_prev)

            @pl.when(jnp.logical_and(k >= 1, k + 2 < _NCHUNK))
            def _(k=k, e_prev=e_prev):
                issue_edges(k + 2, e_prev)

            @pl.when(k + 1 < _NCHUNK)
            def _(e_next=e_next, r_prev=r_prev):
                wait_edges(e_next)
                issue_gather(e_next, r_prev)

        return carry

    lax.fori_loop(0, _NCHUNK // 6, six_body, 0)

    wait_scatter((_NCHUNK - 1) % 3, (_NCHUNK - 1) % 2)
    plsc.subcore_barrier()
    pltpu.sync_copy(agg_sh.at[pl.ds(row0, _RPT)],
                    out_hbm.at[c, pl.ds(row0, _RPT)])


def kernel(x, edge_index, edge_attr, W_self0, W_nbr0, b0, W_self1, W_nbr1, b1):
    src = edge_index[0]
    dst = edge_index[1]
    attr = edge_attr[:, 0]

    perm = jnp.stack(
        [jnp.arange(_D // 2), jnp.arange(_D // 2) + _D // 2], axis=1
    ).reshape(-1)
    W_nbr0 = W_nbr0[:, perm]
    W_nbr1 = W_nbr1[:, perm]

    pad_pos = (jnp.arange(_NW)[:, None] * 997
               + jnp.arange(_EPW2 - _EPW)[None, :] * 131) % _N
    pad_idx = pad_pos.astype(jnp.int32)
    src_p = jnp.concatenate([src.reshape(_NW, _EPW), pad_idx], axis=1).reshape(-1)
    dst_p = jnp.concatenate([dst.reshape(_NW, _EPW), pad_idx], axis=1).reshape(-1)
    attr_p = jnp.concatenate(
        [attr.reshape(_NW, _EPW),
         jnp.zeros((_NW, _EPW2 - _EPW), jnp.float32)], axis=1).reshape(-1)
    zeros = jnp.zeros((_NP, _D), jnp.float32)

    s0, p0 = _mm2(x, W_self0, W_nbr0)
    agg0 = _sc_scatter(p0, src_p, dst_p, attr_p, zeros)
    s1, p1 = _combine_mm2(s0, agg0, b0, W_self1, W_nbr1)
    agg1 = _sc_scatter(p1, src_p, dst_p, attr_p, zeros)
    return _final(s1, agg1, b1)


# speedup vs baseline: 2.0938x; 2.0938x over previous
import functools

import jax
import jax.numpy as jnp
from jax import lax
from jax.experimental import pallas as pl
from jax.experimental.pallas import tpu as pltpu
from jax.experimental.pallas import tpu_sc as plsc

_N = 10000
_E = 320000
_D = 128

_L = 16
_NC = 2
_NS = 16
_NW = _NC * _NS
_EPW = _E // _NW
_C = 120
_EPW2 = 10080
_NCHUNK = _EPW2 // _C
_NP = 10240
_RPT = _NP // _NS

_BM = 1000


def _mm2_body(x_ref, wa_ref, wb_ref, oa_ref, ob_ref):
    x = x_ref[...]
    oa_ref[...] = jnp.dot(x, wa_ref[...], preferred_element_type=jnp.float32)
    ob_ref[...] = jnp.dot(x, wb_ref[...], preferred_element_type=jnp.float32)


def _mm2(x, wa, wb):
    n = x.shape[0]
    return pl.pallas_call(
        _mm2_body,
        grid=(n // _BM,),
        in_specs=[
            pl.BlockSpec((_BM, _D), lambda i: (i, 0)),
            pl.BlockSpec((_D, _D), lambda i: (0, 0)),
            pl.BlockSpec((_D, _D), lambda i: (0, 0)),
        ],
        out_specs=[
            pl.BlockSpec((_BM, _D), lambda i: (i, 0)),
            pl.BlockSpec((_BM, _D), lambda i: (i, 0)),
        ],
        out_shape=[jax.ShapeDtypeStruct((n, _D), jnp.float32)] * 2,
    )(x, wa, wb)


def _combine_mm2_body(s_ref, agg_ref, b_ref, wa_ref, wb_ref, oa_ref, ob_ref):
    h = s_ref[...] + agg_ref[0] + agg_ref[1] + b_ref[...]
    h = jnp.maximum(h, 0.0)
    oa_ref[...] = jnp.dot(h, wa_ref[...], preferred_element_type=jnp.float32)
    ob_ref[...] = jnp.dot(h, wb_ref[...], preferred_element_type=jnp.float32)


def _combine_mm2(s, agg, b, wa, wb):
    n = s.shape[0]
    return pl.pallas_call(
        _combine_mm2_body,
        grid=(n // _BM,),
        in_specs=[
            pl.BlockSpec((_BM, _D), lambda i: (i, 0)),
            pl.BlockSpec((2, _BM, _D), lambda i: (0, i, 0)),
            pl.BlockSpec((1, _D), lambda i: (0, 0)),
            pl.BlockSpec((_D, _D), lambda i: (0, 0)),
            pl.BlockSpec((_D, _D), lambda i: (0, 0)),
        ],
        out_specs=[
            pl.BlockSpec((_BM, _D), lambda i: (i, 0)),
            pl.BlockSpec((_BM, _D), lambda i: (i, 0)),
        ],
        out_shape=[jax.ShapeDtypeStruct((n, _D), jnp.float32)] * 2,
    )(s, agg, b.reshape(1, _D), wa, wb)


def _final_body(s_ref, agg_ref, b_ref, o_ref):
    o_ref[...] = s_ref[...] + agg_ref[0] + agg_ref[1] + b_ref[...]


def _final(s, agg, b):
    n = s.shape[0]
    return pl.pallas_call(
        _final_body,
        grid=(n // _BM,),
        in_specs=[
            pl.BlockSpec((_BM, _D), lambda i: (i, 0)),
            pl.BlockSpec((2, _BM, _D), lambda i: (0, i, 0)),
            pl.BlockSpec((1, _D), lambda i: (0, 0)),
        ],
        out_specs=pl.BlockSpec((_BM, _D), lambda i: (i, 0)),
        out_shape=jax.ShapeDtypeStruct((n, _D), jnp.float32),
    )(s, agg, b.reshape(1, _D))


_mesh = plsc.VectorSubcoreMesh(core_axis_name="c", subcore_axis_name="s")


@functools.partial(
    pl.kernel,
    mesh=_mesh,
    compiler_params=pltpu.CompilerParams(needs_layout_passes=False),
    out_type=jax.ShapeDtypeStruct((2, _NP, _D), jnp.float32),
    scratch_types=[
        pltpu.VMEM((4, _C), jnp.int32),
        pltpu.VMEM((4, _C), jnp.int32),
        pltpu.VMEM((4, _C), jnp.float32),
        pltpu.VMEM((_C, _D), jnp.float32),
        pltpu.VMEM((_C, _D), jnp.float32),
        pltpu.VMEM((_C, _D), jnp.float32),
        pltpu.VMEM_SHARED((_NP, _D), jnp.float32),
        pltpu.SemaphoreType.DMA,
        pltpu.SemaphoreType.DMA,
        pltpu.SemaphoreType.DMA,
        pltpu.SemaphoreType.DMA,
        pltpu.SemaphoreType.DMA,
        pltpu.SemaphoreType.DMA,
        pltpu.SemaphoreType.DMA,
        pltpu.SemaphoreType.DMA,
        pltpu.SemaphoreType.DMA,
        pltpu.SemaphoreType.DMA,
    ],
)
def _sc_scatter(p_hbm, src_hbm, dst_hbm, attr_hbm, zeros_hbm, out_hbm,
                srcr, dstr, attrr, rows0, rows1, rows2, agg_sh,
                esem0, esem1, esem2, esem3,
                gsem0, gsem1, gsem2, ssem0, ssem1, ssem2):
    c = lax.axis_index("c")
    s = lax.axis_index("s")
    wid = s * _NC + c
    row0 = s * _RPT

    pltpu.sync_copy(zeros_hbm.at[pl.ds(row0, _RPT)], agg_sh.at[pl.ds(row0, _RPT)])
    plsc.subcore_barrier()

    base = pl.multiple_of(wid * _EPW2, 8)
    esem = (esem0, esem1, esem2, esem3)
    rows = (rows0, rows1, rows2)
    gsem = (gsem0, gsem1, gsem2)
    ssem = (ssem0, ssem1, ssem2)

    def issue_edges(ck, e):
        off = pl.multiple_of(base + ck * _C, 8)
        pltpu.async_copy(src_hbm.at[pl.ds(off, _C)], srcr.at[e], esem[e])
        pltpu.async_copy(dst_hbm.at[pl.ds(off, _C)], dstr.at[e], esem[e])
        pltpu.async_copy(attr_hbm.at[pl.ds(off, _C)], attrr.at[e], esem[e])

    def wait_edges(e):
        pltpu.make_async_copy(src_hbm.at[pl.ds(0, _C)], srcr.at[e], esem[e]).wait()
        pltpu.make_async_copy(dst_hbm.at[pl.ds(0, _C)], dstr.at[e], esem[e]).wait()
        pltpu.make_async_copy(attr_hbm.at[pl.ds(0, _C)], attrr.at[e], esem[e]).wait()

    def issue_gather(e, r):
        pltpu.async_copy(p_hbm.at[srcr.at[e]], rows[r], gsem[r])

    def wait_gather(e, r):
        pltpu.make_async_copy(p_hbm.at[srcr.at[e]], rows[r], gsem[r]).wait()

    def issue_scatter(e, r):
        pltpu.async_copy(rows[r], agg_sh.at[dstr.at[e]], ssem[r], add=True)

    def wait_scatter(e, r):
        pltpu.make_async_copy(rows[r], agg_sh.at[dstr.at[e]], ssem[r]).wait()

    def scale(e, r):
        av = attrr.at[e]
        rv = rows[r]

        def row_body(i, rcarry):
            a = plsc.load_gather(av, [jnp.full((_L,), i, jnp.int32)])
            for j in range(_D // _L):
                sl = pl.ds(j * _L, _L)
                rv[i, sl] = rv[i, sl] * a
            return rcarry

        lax.fori_loop(0, _C, row_body, 0, unroll=2)

    issue_edges(0, 0)
    issue_edges(1, 1)
    issue_edges(2, 2)
    issue_edges(3, 3)
    wait_edges(0)
    issue_gather(0, 0)

    def twelve_body(t, carry):
        k12 = t * 12
        for j in range(12):
            e = j % 4
            r = j % 3
            e1 = (j + 1) % 4
            e2 = (j + 2) % 4
            r1 = (j + 1) % 3
            k = k12 + j

            wait_gather(e, r)

            @pl.when(k >= 2)
            def _(e2=e2, r1=r1):
                wait_scatter(e2, r1)

            @pl.when(k + 1 < _NCHUNK)
            def _(e1=e1, r1=r1, k=k):
                wait_edges(e1)
                issue_gather(e1, r1)

            @pl.when(jnp.logical_and(k >= 2, k + 2 < _NCHUNK))
            def _(k=k, e2=e2):
                issue_edges(k + 2, e2)

            scale(e, r)
            issue_scatter(e, r)
        return carry

    lax.fori_loop(0, _NCHUNK // 12, twelve_body, 0)

    wait_scatter((_NCHUNK - 2) % 4, (_NCHUNK - 2) % 3)
    wait_scatter((_NCHUNK - 1) % 4, (_NCHUNK - 1) % 3)
    plsc.subcore_barrier()
    pltpu.sync_copy(agg_sh.at[pl.ds(row0, _RPT)],
                    out_hbm.at[c, pl.ds(row0, _RPT)])


def kernel(x, edge_index, edge_attr, W_self0, W_nbr0, b0, W_self1, W_nbr1, b1):
    src = edge_index[0]
    dst = edge_index[1]
    attr = edge_attr[:, 0]

    pad_pos = (jnp.arange(_NW)[:, None] * 997
               + jnp.arange(_EPW2 - _EPW)[None, :] * 131) % _N
    pad_idx = pad_pos.astype(jnp.int32)
    src_p = jnp.concatenate([src.reshape(_NW, _EPW), pad_idx], axis=1).reshape(-1)
    dst_p = jnp.concatenate([dst.reshape(_NW, _EPW), pad_idx], axis=1).reshape(-1)
    attr_p = jnp.concatenate(
        [attr.reshape(_NW, _EPW),
         jnp.zeros((_NW, _EPW2 - _EPW), jnp.float32)], axis=1).reshape(-1)
    zeros = jnp.zeros((_NP, _D), jnp.float32)

    s0, p0 = _mm2(x, W_self0, W_nbr0)
    agg0 = _sc_scatter(p0, src_p, dst_p, attr_p, zeros)
    s1, p1 = _combine_mm2(s0, agg0, b0, W_self1, W_nbr1)
    agg1 = _sc_scatter(p1, src_p, dst_p, attr_p, zeros)
    return _final(s1, agg1, b1)


# speedup vs baseline: 2.1088x; 1.0072x over previous
import functools

import jax
import jax.numpy as jnp
from jax import lax
from jax.experimental import pallas as pl
from jax.experimental.pallas import tpu as pltpu
from jax.experimental.pallas import tpu_sc as plsc

_N = 10000
_E = 320000
_D = 128

_L = 16
_NC = 2
_NS = 16
_NW = _NC * _NS
_EPW = _E // _NW
_C = 120
_EPW2 = 10080
_NCHUNK = _EPW2 // _C
_NP = 10240
_RPT = _NP // _NS

_BM = 1000


def _mm2_body(x_ref, wa_ref, wb_ref, oa_ref, ob_ref):
    x = x_ref[...]
    oa_ref[...] = jnp.dot(x, wa_ref[...], preferred_element_type=jnp.float32)
    ob_ref[...] = jnp.dot(x, wb_ref[...], preferred_element_type=jnp.float32)


def _mm2(x, wa, wb):
    n = x.shape[0]
    return pl.pallas_call(
        _mm2_body,
        grid=(n // _BM,),
        in_specs=[
            pl.BlockSpec((_BM, _D), lambda i: (i, 0)),
            pl.BlockSpec((_D, _D), lambda i: (0, 0)),
            pl.BlockSpec((_D, _D), lambda i: (0, 0)),
        ],
        out_specs=[
            pl.BlockSpec((_BM, _D), lambda i: (i, 0)),
            pl.BlockSpec((_BM, _D), lambda i: (i, 0)),
        ],
        out_shape=[jax.ShapeDtypeStruct((n, _D), jnp.float32)] * 2,
    )(x, wa, wb)


def _combine_mm2_body(s_ref, agg_ref, b_ref, wa_ref, wb_ref, oa_ref, ob_ref):
    h = s_ref[...] + agg_ref[0] + agg_ref[1] + b_ref[...]
    h = jnp.maximum(h, 0.0)
    oa_ref[...] = jnp.dot(h, wa_ref[...], preferred_element_type=jnp.float32)
    ob_ref[...] = jnp.dot(h, wb_ref[...], preferred_element_type=jnp.float32)


def _combine_mm2(s, agg, b, wa, wb):
    n = s.shape[0]
    return pl.pallas_call(
        _combine_mm2_body,
        grid=(n // _BM,),
        in_specs=[
            pl.BlockSpec((_BM, _D), lambda i: (i, 0)),
            pl.BlockSpec((2, _BM, _D), lambda i: (0, i, 0)),
            pl.BlockSpec((1, _D), lambda i: (0, 0)),
            pl.BlockSpec((_D, _D), lambda i: (0, 0)),
            pl.BlockSpec((_D, _D), lambda i: (0, 0)),
        ],
        out_specs=[
            pl.BlockSpec((_BM, _D), lambda i: (i, 0)),
            pl.BlockSpec((_BM, _D), lambda i: (i, 0)),
        ],
        out_shape=[jax.ShapeDtypeStruct((n, _D), jnp.float32)] * 2,
    )(s, agg, b.reshape(1, _D), wa, wb)


def _final_body(s_ref, agg_ref, b_ref, o_ref):
    o_ref[...] = s_ref[...] + agg_ref[0] + agg_ref[1] + b_ref[...]


def _final(s, agg, b):
    n = s.shape[0]
    return pl.pallas_call(
        _final_body,
        grid=(n // _BM,),
        in_specs=[
            pl.BlockSpec((_BM, _D), lambda i: (i, 0)),
            pl.BlockSpec((2, _BM, _D), lambda i: (0, i, 0)),
            pl.BlockSpec((1, _D), lambda i: (0, 0)),
        ],
        out_specs=pl.BlockSpec((_BM, _D), lambda i: (i, 0)),
        out_shape=jax.ShapeDtypeStruct((n, _D), jnp.float32),
    )(s, agg, b.reshape(1, _D))


_mesh = plsc.VectorSubcoreMesh(core_axis_name="c", subcore_axis_name="s")


@functools.partial(
    pl.kernel,
    mesh=_mesh,
    compiler_params=pltpu.CompilerParams(needs_layout_passes=False),
    out_type=jax.ShapeDtypeStruct((2, _NP, _D), jnp.float32),
    scratch_types=[
        pltpu.VMEM((4, _C), jnp.int32),
        pltpu.VMEM((4, _C), jnp.int32),
        pltpu.VMEM((4, _C), jnp.float32),
        pltpu.VMEM((_C, _D), jnp.float32),
        pltpu.VMEM((_C, _D), jnp.float32),
        pltpu.VMEM((_C, _D), jnp.float32),
        pltpu.VMEM_SHARED((_NP, _D), jnp.float32),
        pltpu.SemaphoreType.DMA,
        pltpu.SemaphoreType.DMA,
        pltpu.SemaphoreType.DMA,
        pltpu.SemaphoreType.DMA,
        pltpu.SemaphoreType.DMA,
        pltpu.SemaphoreType.DMA,
        pltpu.SemaphoreType.DMA,
        pltpu.SemaphoreType.DMA,
        pltpu.SemaphoreType.DMA,
        pltpu.SemaphoreType.DMA,
    ],
)
def _sc_scatter(p_hbm, src_hbm, dst_hbm, attr_hbm, zeros_hbm, out_hbm,
                srcr, dstr, attrr, rows0, rows1, rows2, agg_sh,
                esem0, esem1, esem2, esem3,
                gsem0, gsem1, gsem2, ssem0, ssem1, ssem2):
    c = lax.axis_index("c")
    s = lax.axis_index("s")
    wid = s * _NC + c
    row0 = s * _RPT

    pltpu.sync_copy(zeros_hbm.at[pl.ds(row0, _RPT)], agg_sh.at[pl.ds(row0, _RPT)])
    plsc.subcore_barrier()

    base = pl.multiple_of(wid * _EPW2, 8)
    esem = (esem0, esem1, esem2, esem3)
    rows = (rows0, rows1, rows2)
    gsem = (gsem0, gsem1, gsem2)
    ssem = (ssem0, ssem1, ssem2)

    def issue_edges(ck, e):
        off = pl.multiple_of(base + ck * _C, 8)
        pltpu.async_copy(src_hbm.at[pl.ds(off, _C)], srcr.at[e], esem[e])
        pltpu.async_copy(dst_hbm.at[pl.ds(off, _C)], dstr.at[e], esem[e])
        pltpu.async_copy(attr_hbm.at[pl.ds(off, _C)], attrr.at[e], esem[e])

    def wait_edges(e):
        pltpu.make_async_copy(src_hbm.at[pl.ds(0, _C)], srcr.at[e], esem[e]).wait()
        pltpu.make_async_copy(dst_hbm.at[pl.ds(0, _C)], dstr.at[e], esem[e]).wait()
        pltpu.make_async_copy(attr_hbm.at[pl.ds(0, _C)], attrr.at[e], esem[e]).wait()

    _G = _C // 3

    def issue_gather(e, r):
        for h in range(3):
            pltpu.async_copy(p_hbm.at[srcr.at[e, pl.ds(h * _G, _G)]],
                             rows[r].at[pl.ds(h * _G, _G)], gsem[r])

    def wait_gather(e, r):
        for h in range(3):
            pltpu.make_async_copy(p_hbm.at[srcr.at[e, pl.ds(h * _G, _G)]],
                                  rows[r].at[pl.ds(h * _G, _G)], gsem[r]).wait()

    def issue_scatter(e, r):
        pltpu.async_copy(rows[r], agg_sh.at[dstr.at[e]], ssem[r], add=True)

    def wait_scatter(e, r):
        pltpu.make_async_copy(rows[r], agg_sh.at[dstr.at[e]], ssem[r]).wait()

    def scale(e, r):
        av = attrr.at[e]
        rv = rows[r]

        def row_body(i, rcarry):
            a = plsc.load_gather(av, [jnp.full((_L,), i, jnp.int32)])
            for j in range(_D // _L):
                sl = pl.ds(j * _L, _L)
                rv[i, sl] = rv[i, sl] * a
            return rcarry

        lax.fori_loop(0, _C, row_body, 0, unroll=2)

    issue_edges(0, 0)
    issue_edges(1, 1)
    issue_edges(2, 2)
    issue_edges(3, 3)
    wait_edges(0)
    issue_gather(0, 0)

    def twelve_body(t, carry):
        k12 = t * 12
        for j in range(12):
            e = j % 4
            r = j % 3
            e1 = (j + 1) % 4
            e2 = (j + 2) % 4
            r1 = (j + 1) % 3
            k = k12 + j

            wait_gather(e, r)

            @pl.when(k >= 2)
            def _(e2=e2, r1=r1):
                wait_scatter(e2, r1)

            @pl.when(k + 1 < _NCHUNK)
            def _(e1=e1, r1=r1, k=k):
                wait_edges(e1)
                issue_gather(e1, r1)

            @pl.when(jnp.logical_and(k >= 2, k + 2 < _NCHUNK))
            def _(k=k, e2=e2):
                issue_edges(k + 2, e2)

            scale(e, r)
            issue_scatter(e, r)
        return carry

    lax.fori_loop(0, _NCHUNK // 12, twelve_body, 0)

    wait_scatter((_NCHUNK - 2) % 4, (_NCHUNK - 2) % 3)
    wait_scatter((_NCHUNK - 1) % 4, (_NCHUNK - 1) % 3)
    plsc.subcore_barrier()
    pltpu.sync_copy(agg_sh.at[pl.ds(row0, _RPT)],
                    out_hbm.at[c, pl.ds(row0, _RPT)])


def kernel(x, edge_index, edge_attr, W_self0, W_nbr0, b0, W_self1, W_nbr1, b1):
    src = edge_index[0]
    dst = edge_index[1]
    attr = edge_attr[:, 0]

    pad_pos = (jnp.arange(_NW)[:, None] * 997
               + jnp.arange(_EPW2 - _EPW)[None, :] * 131) % _N
    pad_idx = pad_pos.astype(jnp.int32)
    src_p = jnp.concatenate([src.reshape(_NW, _EPW), pad_idx], axis=1).reshape(-1)
    dst_p = jnp.concatenate([dst.reshape(_NW, _EPW), pad_idx], axis=1).reshape(-1)
    attr_p = jnp.concatenate(
        [attr.reshape(_NW, _EPW),
         jnp.zeros((_NW, _EPW2 - _EPW), jnp.float32)], axis=1).reshape(-1)
    zeros = jnp.zeros((_NP, _D), jnp.float32)

    s0, p0 = _mm2(x, W_self0, W_nbr0)
    agg0 = _sc_scatter(p0, src_p, dst_p, attr_p, zeros)
    s1, p1 = _combine_mm2(s0, agg0, b0, W_self1, W_nbr1)
    agg1 = _sc_scatter(p1, src_p, dst_p, attr_p, zeros)
    return _final(s1, agg1, b1)
